# Initial kernel scaffold; baseline (speedup 1.0000x reference)
import jax
import jax.numpy as jnp
from jax.experimental import pallas as pl


def kernel(xyz1, xyz2, points1, points2, conv0_w, conv0_b, conv1_w, conv1_b, wn1_w0, wn1_b0, wn1_w1, wn1_b1, wn1_w2, wn1_b2, wn2_w0, wn2_b0, wn2_w1, wn2_b1, wn2_w2, wn2_b2):
    raise NotImplementedError("write your pallas kernel here")


# speedup vs baseline: 3.9622x; 3.9622x over previous
import functools

import jax
import jax.numpy as jnp
from jax import lax
from jax.experimental import pallas as pl
from jax.experimental.pallas import tpu as pltpu
from jax.experimental.pallas import tpu_sc as plsc

K = 16
LEAK = 0.1
TQ_KNN = 256
TQ_MLP = 512
SC_NC = 2
SC_NS = 16
SC_NW = SC_NC * SC_NS
IMAX = 0x7FFFFFFF


def _knn_body(q_ref, x_ref, out_ref):
    tq = out_ref.shape[0]
    q = q_ref[...]
    x = x_ref[...]
    nx = x.shape[1]
    g = nx // 128
    xsq = jnp.sum(x * x, axis=0, keepdims=True)
    qsq = jnp.sum(q * q, axis=0)
    qx = lax.dot_general(q.astype(jnp.bfloat16), x.astype(jnp.bfloat16),
                         (((0,), (0,)), ((), ())),
                         preferred_element_type=jnp.float32)
    d = (-2.0 * qx + qsq[:, None]) + xsq
    b = lax.bitcast_convert_type(d, jnp.int32)
    key = b ^ (lax.shift_right_arithmetic(b, 31) & IMAX)
    k3 = key.reshape(tq, g, 128)
    flat = lax.broadcasted_iota(jnp.int32, (tq, g, 128), 1) * 128 \
        + lax.broadcasted_iota(jnp.int32, (tq, g, 128), 2)
    prev = jnp.full((tq, 1), -2147483648, jnp.int32)
    cols = []
    for _ in range(K):
        mg = jnp.min(jnp.where(k3 > prev[:, :, None], k3, IMAX), axis=2)
        kmin = jnp.min(mg, axis=1, keepdims=True)
        idx = jnp.min(jnp.min(jnp.where(k3 == kmin[:, :, None], flat, nx),
                              axis=2), axis=1, keepdims=True)
        cols.append(idx)
        prev = kmin
    out_ref[...] = jnp.concatenate(cols, axis=1)


def _knn(q8, x8):
    nq = q8.shape[1]
    nx = x8.shape[1]
    return pl.pallas_call(
        _knn_body,
        grid=(nq // TQ_KNN,),
        in_specs=[pl.BlockSpec((8, TQ_KNN), lambda i: (0, i)),
                  pl.BlockSpec((8, nx), lambda i: (0, 0))],
        out_specs=pl.BlockSpec((TQ_KNN, K), lambda i: (i, 0)),
        out_shape=jax.ShapeDtypeStruct((nq, K), jnp.int32),
    )(q8, x8)


def _sc_gather(table, idx_flat):
    b = idx_flat.shape[0]
    d = table.shape[1]
    r = b // 128
    rw = r // SC_NW
    nb = 4
    ng = rw // nb
    idx2d = idx_flat.reshape(r, 128)
    mesh = plsc.VectorSubcoreMesh(core_axis_name="c", subcore_axis_name="s")

    @functools.partial(
        pl.kernel, mesh=mesh,
        out_type=jax.ShapeDtypeStruct((r, 128, d), jnp.float32),
        compiler_params=pltpu.CompilerParams(use_tc_tiling_on_sc=False),
        scratch_types=[pltpu.VMEM((rw, 128), jnp.int32),
                       pltpu.VMEM((nb, 128, d), jnp.float32),
                       pltpu.SemaphoreType.DMA],
    )
    def gk(table_hbm, idx_hbm, out_hbm, idx_v, rows_v, sem):
        wid = lax.axis_index("s") * SC_NC + lax.axis_index("c")
        r0 = wid * rw
        pltpu.sync_copy(idx_hbm.at[pl.ds(r0, rw)], idx_v)

        def body(gi, carry):
            base = gi * nb
            cps = [pltpu.async_copy(table_hbm.at[idx_v.at[base + j]],
                                    rows_v.at[j], sem)
                   for j in range(nb)]
            for cp in cps:
                cp.wait()
            pltpu.sync_copy(rows_v, out_hbm.at[pl.ds(r0 + base, nb)])
            return carry

        lax.fori_loop(0, ng, body, 0)

    return gk(table, idx2d).reshape(b, d)


def _leaky(v):
    return jnp.where(v >= 0, v, LEAK * v)


def _bdot(a, b):
    return jnp.dot(a.astype(jnp.bfloat16), b.astype(jnp.bfloat16),
                   preferred_element_type=jnp.float32)


def _stage1_body(g_ref, gx_ref, qrow_ref, p1_ref, wp2_ref, wd16_ref, wp1_ref,
                 b0_ref, w1_ref, b1_ref, n0g_ref, nb0_ref,
                 n1_ref, nb1_ref, n2_ref, nb2_ref, out_ref):
    tq = out_ref.shape[0]
    g = g_ref[...].reshape(tq * K, 64)
    gx = gx_ref[...]
    qrow = qrow_ref[...]
    dirp = (gx - qrow[:, None, :]).reshape(tq * K, 16)
    t0 = _bdot(p1_ref[...], wp1_ref[...]) + b0_ref[...]
    z0 = ((_bdot(g, wp2_ref[...]) + _bdot(dirp, wd16_ref[...]))
          .reshape(tq, K, 64) + t0[:, None, :])
    a0 = _leaky(z0).reshape(tq * K, 64)
    z1 = _bdot(a0, w1_ref[...]) + b1_ref[...]
    a1 = _leaky(z1).reshape(tq, K, 64)
    u0 = jnp.maximum(_bdot(dirp, n0g_ref[...]) + nb0_ref[...], 0.0)
    u1 = jnp.maximum(_bdot(u0, n1_ref[...]) + nb1_ref[...], 0.0)
    w = jnp.maximum(_bdot(u1, n2_ref[...]) + nb2_ref[...], 0.0).reshape(tq, K, 64)
    out_ref[...] = jnp.sum(w * a1, axis=1)


def _full(shape):
    return pl.BlockSpec(shape, lambda i: tuple(0 for _ in shape))


def _stage1(g13, gx13, qrow, p1r, wp2, wd16, wp1, b0r, w1, b1r,
            n0g, nb0, n1, nb1, n2, nb2):
    n1q = p1r.shape[0]
    return pl.pallas_call(
        _stage1_body,
        grid=(n1q // TQ_MLP,),
        in_specs=[pl.BlockSpec((TQ_MLP, K, 64), lambda i: (i, 0, 0)),
                  pl.BlockSpec((TQ_MLP, K, 16), lambda i: (i, 0, 0)),
                  pl.BlockSpec((TQ_MLP, 16), lambda i: (i, 0)),
                  pl.BlockSpec((TQ_MLP, 64), lambda i: (i, 0)),
                  _full((64, 64)), _full((16, 64)), _full((64, 64)),
                  _full((1, 64)), _full((64, 64)),
                  _full((1, 64)), _full((16, 8)), _full((1, 8)),
                  _full((8, 8)), _full((1, 8)), _full((8, 64)), _full((1, 64))],
        out_specs=pl.BlockSpec((TQ_MLP, 64), lambda i: (i, 0)),
        out_shape=jax.ShapeDtypeStruct((n1q, 64), jnp.float32),
    )(g13, gx13, qrow, p1r, wp2, wd16, wp1, b0r, w1, b1r,
      n0g, nb0, n1, nb1, n2, nb2)


def _stage2_body(gx_ref, gc_ref, qrow_ref, m0g_ref, mb0_ref,
                 m1_ref, mb1_ref, m2_ref, mb2_ref, out_ref):
    tq = out_ref.shape[0]
    gx = gx_ref[...]
    qrow = qrow_ref[...]
    dirp = (gx - qrow[:, None, :]).reshape(tq * K, 16)
    u0 = jnp.maximum(_bdot(dirp, m0g_ref[...]) + mb0_ref[...], 0.0)
    u1 = jnp.maximum(_bdot(u0, m1_ref[...]) + mb1_ref[...], 0.0)
    w = jnp.maximum(_bdot(u1, m2_ref[...]) + mb2_ref[...], 0.0).reshape(tq, K, 64)
    out_ref[...] = jnp.sum(w * gc_ref[...], axis=1)


def _stage2(gx3, gc3, qrow, m0g, mb0, m1, mb1, m2, mb2):
    n1q = qrow.shape[0]
    return pl.pallas_call(
        _stage2_body,
        grid=(n1q // TQ_MLP,),
        in_specs=[pl.BlockSpec((TQ_MLP, K, 16), lambda i: (i, 0, 0)),
                  pl.BlockSpec((TQ_MLP, K, 64), lambda i: (i, 0, 0)),
                  pl.BlockSpec((TQ_MLP, 16), lambda i: (i, 0)),
                  _full((16, 8)), _full((1, 8)),
                  _full((8, 8)), _full((1, 8)), _full((8, 64)), _full((1, 64))],
        out_specs=pl.BlockSpec((TQ_MLP, 64), lambda i: (i, 0)),
        out_shape=jax.ShapeDtypeStruct((n1q, 64), jnp.float32),
    )(gx3, gc3, qrow, m0g, mb0, m1, mb1, m2, mb2)


def kernel(xyz1, xyz2, points1, points2, conv0_w, conv0_b, conv1_w, conv1_b,
           wn1_w0, wn1_b0, wn1_w1, wn1_b1, wn1_w2, wn1_b2,
           wn2_w0, wn2_b0, wn2_w1, wn2_b1, wn2_w2, wn2_b2):
    n1 = xyz1.shape[2]
    n2 = xyz2.shape[2]
    f32 = jnp.float32
    x1t = xyz1[0]
    x2t = xyz2[0]
    p1r = jnp.transpose(points1[0])
    p2r = jnp.transpose(points2[0])
    x1p8 = jnp.concatenate([x1t, jnp.zeros((5, n1), f32)], axis=0)
    x2p8 = jnp.concatenate([x2t, jnp.zeros((5, n2), f32)], axis=0)

    idx1 = _knn(x1p8, x2p8)
    idx2 = _knn(x1p8, x1p8)

    table_x2 = jnp.concatenate(
        [jnp.transpose(x2t), jnp.zeros((n2, 13), f32)], axis=1)
    table_x = jnp.concatenate(
        [jnp.transpose(x1t), jnp.zeros((n1, 13), f32)], axis=1)

    g1p = _sc_gather(p2r, idx1.reshape(-1))
    g1x = _sc_gather(table_x2, idx1.reshape(-1))
    gx2 = _sc_gather(table_x, idx2.reshape(-1))

    wd3t = jnp.transpose(conv0_w[:, 128:131])
    wd16 = jnp.concatenate([wd3t, jnp.zeros((13, 64), f32)], axis=0)
    wp2 = jnp.transpose(conv0_w[:, 64:128])
    wp1 = jnp.transpose(conv0_w[:, :64])
    w1 = jnp.transpose(conv1_w)

    def wn_prep(w0, w1_, w2):
        w0t = jnp.transpose(w0)
        n0g = jnp.concatenate([w0t, jnp.zeros((13, 8), f32)], axis=0)
        return n0g, jnp.transpose(w1_), jnp.transpose(w2)

    n0g, n1w, n2w = wn_prep(wn1_w0, wn1_w1, wn1_w2)
    m0g, m1w, m2w = wn_prep(wn2_w0, wn2_w1, wn2_w2)

    ppc = _stage1(g1p.reshape(n1, K, 64), g1x.reshape(n1, K, 16), table_x, p1r,
                  wp2, wd16, wp1, conv0_b[None], w1, conv1_b[None],
                  n0g, wn1_b0[None], n1w, wn1_b1[None],
                  n2w, wn1_b2[None])

    gc = _sc_gather(ppc, idx2.reshape(-1))

    out = _stage2(gx2.reshape(n1, K, 16), gc.reshape(n1, K, 64), table_x,
                  m0g, wn2_b0[None], m1w, wn2_b1[None],
                  m2w, wn2_b2[None])
    return jnp.transpose(out)[None]


# speedup vs baseline: 10.0409x; 2.5342x over previous
import functools

import jax
import jax.numpy as jnp
from jax import lax
from jax.experimental import pallas as pl
from jax.experimental.pallas import tpu as pltpu
from jax.experimental.pallas import tpu_sc as plsc

K = 16
LEAK = 0.1
TQ_KNN = 256
TQ_MLP = 512
SC_NC = 2
SC_NS = 16
SC_NW = SC_NC * SC_NS
IMAX = 0x7FFFFFFF


def _knn_body(q_ref, x_ref, out_ref):
    tq = out_ref.shape[0]
    q = q_ref[...]
    x = x_ref[...]
    nx = x.shape[1]
    g = nx // 128
    xsq = jnp.sum(x * x, axis=0, keepdims=True)
    qsq = jnp.sum(q * q, axis=0)
    qx = lax.dot_general(q.astype(jnp.bfloat16), x.astype(jnp.bfloat16),
                         (((0,), (0,)), ((), ())),
                         preferred_element_type=jnp.float32)
    d = (-2.0 * qx + qsq[:, None]) + xsq
    b = lax.bitcast_convert_type(d, jnp.int32)
    key = b ^ (lax.shift_right_arithmetic(b, 31) & IMAX)
    k3 = key.reshape(tq, g, 128)
    giota = lax.broadcasted_iota(jnp.int32, (tq, g, 128), 1)
    liota = lax.broadcasted_iota(jnp.int32, (tq, 128), 1)
    prev = jnp.full((tq, 1), -2147483648, jnp.int32)
    cols = []
    for _ in range(K):
        v = jnp.where(k3 > prev[:, :, None], k3, IMAX)
        gi = giota
        size = g
        while size > 1:
            half = size // 2
            av, bv = v[:, :half], v[:, half:]
            cond = av <= bv
            v = jnp.where(cond, av, bv)
            gi = jnp.where(cond, gi[:, :half], gi[:, half:])
            size = half
        v2 = v[:, 0]
        g2 = gi[:, 0]
        kmin = jnp.min(v2, axis=1, keepdims=True)
        flat = jnp.where(v2 == kmin, g2 * 128 + liota, nx)
        idx = jnp.min(flat, axis=1, keepdims=True)
        cols.append(idx)
        prev = kmin
    out_ref[...] = jnp.concatenate(cols, axis=1)


def _knn(q8, x8):
    nq = q8.shape[1]
    nx = x8.shape[1]
    return pl.pallas_call(
        _knn_body,
        grid=(nq // TQ_KNN,),
        in_specs=[pl.BlockSpec((8, TQ_KNN), lambda i: (0, i)),
                  pl.BlockSpec((8, nx), lambda i: (0, 0))],
        out_specs=pl.BlockSpec((TQ_KNN, K), lambda i: (i, 0)),
        out_shape=jax.ShapeDtypeStruct((nq, K), jnp.int32),
    )(q8, x8)


def _sc_gather(table, idx_flat):
    b = idx_flat.shape[0]
    d = table.shape[1]
    r = b // 128
    rw = r // SC_NW
    nb = 4
    ng = rw // nb
    idx2d = idx_flat.reshape(r, 128)
    mesh = plsc.VectorSubcoreMesh(core_axis_name="c", subcore_axis_name="s")

    @functools.partial(
        pl.kernel, mesh=mesh,
        out_type=jax.ShapeDtypeStruct((r, 128, d), jnp.float32),
        compiler_params=pltpu.CompilerParams(use_tc_tiling_on_sc=False),
        scratch_types=[pltpu.VMEM((rw, 128), jnp.int32),
                       pltpu.VMEM((nb, 128, d), jnp.float32),
                       pltpu.SemaphoreType.DMA],
    )
    def gk(table_hbm, idx_hbm, out_hbm, idx_v, rows_v, sem):
        wid = lax.axis_index("s") * SC_NC + lax.axis_index("c")
        r0 = wid * rw
        pltpu.sync_copy(idx_hbm.at[pl.ds(r0, rw)], idx_v)

        def body(gi, carry):
            base = gi * nb
            cps = [pltpu.async_copy(table_hbm.at[idx_v.at[base + j]],
                                    rows_v.at[j], sem)
                   for j in range(nb)]
            for cp in cps:
                cp.wait()
            pltpu.sync_copy(rows_v, out_hbm.at[pl.ds(r0 + base, nb)])
            return carry

        lax.fori_loop(0, ng, body, 0)

    return gk(table, idx2d).reshape(b, d)


def _leaky(v):
    return jnp.where(v >= 0, v, LEAK * v)


def _bdot(a, b):
    return jnp.dot(a.astype(jnp.bfloat16), b.astype(jnp.bfloat16),
                   preferred_element_type=jnp.float32)


def _stage1_body(g_ref, gx_ref, qrow_ref, p1_ref, wp2_ref, wd16_ref, wp1_ref,
                 b0_ref, w1_ref, b1_ref, n0g_ref, nb0_ref,
                 n1_ref, nb1_ref, n2_ref, nb2_ref, out_ref):
    tq = out_ref.shape[0]
    g = g_ref[...].reshape(tq * K, 64)
    gx = gx_ref[...]
    qrow = qrow_ref[...]
    dirp = (gx - qrow[:, None, :]).reshape(tq * K, 16)
    t0 = _bdot(p1_ref[...], wp1_ref[...]) + b0_ref[...]
    z0 = ((_bdot(g, wp2_ref[...]) + _bdot(dirp, wd16_ref[...]))
          .reshape(tq, K, 64) + t0[:, None, :])
    a0 = _leaky(z0).reshape(tq * K, 64)
    z1 = _bdot(a0, w1_ref[...]) + b1_ref[...]
    a1 = _leaky(z1).reshape(tq, K, 64)
    u0 = jnp.maximum(_bdot(dirp, n0g_ref[...]) + nb0_ref[...], 0.0)
    u1 = jnp.maximum(_bdot(u0, n1_ref[...]) + nb1_ref[...], 0.0)
    w = jnp.maximum(_bdot(u1, n2_ref[...]) + nb2_ref[...], 0.0).reshape(tq, K, 64)
    out_ref[...] = jnp.sum(w * a1, axis=1)


def _full(shape):
    return pl.BlockSpec(shape, lambda i: tuple(0 for _ in shape))


def _stage1(g13, gx13, qrow, p1r, wp2, wd16, wp1, b0r, w1, b1r,
            n0g, nb0, n1, nb1, n2, nb2):
    n1q = p1r.shape[0]
    return pl.pallas_call(
        _stage1_body,
        grid=(n1q // TQ_MLP,),
        in_specs=[pl.BlockSpec((TQ_MLP, K, 64), lambda i: (i, 0, 0)),
                  pl.BlockSpec((TQ_MLP, K, 16), lambda i: (i, 0, 0)),
                  pl.BlockSpec((TQ_MLP, 16), lambda i: (i, 0)),
                  pl.BlockSpec((TQ_MLP, 64), lambda i: (i, 0)),
                  _full((64, 64)), _full((16, 64)), _full((64, 64)),
                  _full((1, 64)), _full((64, 64)),
                  _full((1, 64)), _full((16, 8)), _full((1, 8)),
                  _full((8, 8)), _full((1, 8)), _full((8, 64)), _full((1, 64))],
        out_specs=pl.BlockSpec((TQ_MLP, 64), lambda i: (i, 0)),
        out_shape=jax.ShapeDtypeStruct((n1q, 64), jnp.float32),
    )(g13, gx13, qrow, p1r, wp2, wd16, wp1, b0r, w1, b1r,
      n0g, nb0, n1, nb1, n2, nb2)


def _stage2_body(gx_ref, gc_ref, qrow_ref, m0g_ref, mb0_ref,
                 m1_ref, mb1_ref, m2_ref, mb2_ref, out_ref):
    tq = out_ref.shape[0]
    gx = gx_ref[...]
    qrow = qrow_ref[...]
    dirp = (gx - qrow[:, None, :]).reshape(tq * K, 16)
    u0 = jnp.maximum(_bdot(dirp, m0g_ref[...]) + mb0_ref[...], 0.0)
    u1 = jnp.maximum(_bdot(u0, m1_ref[...]) + mb1_ref[...], 0.0)
    w = jnp.maximum(_bdot(u1, m2_ref[...]) + mb2_ref[...], 0.0).reshape(tq, K, 64)
    out_ref[...] = jnp.sum(w * gc_ref[...], axis=1)


def _stage2(gx3, gc3, qrow, m0g, mb0, m1, mb1, m2, mb2):
    n1q = qrow.shape[0]
    return pl.pallas_call(
        _stage2_body,
        grid=(n1q // TQ_MLP,),
        in_specs=[pl.BlockSpec((TQ_MLP, K, 16), lambda i: (i, 0, 0)),
                  pl.BlockSpec((TQ_MLP, K, 64), lambda i: (i, 0, 0)),
                  pl.BlockSpec((TQ_MLP, 16), lambda i: (i, 0)),
                  _full((16, 8)), _full((1, 8)),
                  _full((8, 8)), _full((1, 8)), _full((8, 64)), _full((1, 64))],
        out_specs=pl.BlockSpec((TQ_MLP, 64), lambda i: (i, 0)),
        out_shape=jax.ShapeDtypeStruct((n1q, 64), jnp.float32),
    )(gx3, gc3, qrow, m0g, mb0, m1, mb1, m2, mb2)


def kernel(xyz1, xyz2, points1, points2, conv0_w, conv0_b, conv1_w, conv1_b,
           wn1_w0, wn1_b0, wn1_w1, wn1_b1, wn1_w2, wn1_b2,
           wn2_w0, wn2_b0, wn2_w1, wn2_b1, wn2_w2, wn2_b2):
    n1 = xyz1.shape[2]
    n2 = xyz2.shape[2]
    f32 = jnp.float32
    x1t = xyz1[0]
    x2t = xyz2[0]
    p1r = jnp.transpose(points1[0])
    p2r = jnp.transpose(points2[0])
    x1p8 = jnp.concatenate([x1t, jnp.zeros((5, n1), f32)], axis=0)
    x2p8 = jnp.concatenate([x2t, jnp.zeros((5, n2), f32)], axis=0)

    idx1 = _knn(x1p8, x2p8)
    idx2 = _knn(x1p8, x1p8)

    table_x2 = jnp.concatenate(
        [jnp.transpose(x2t), jnp.zeros((n2, 13), f32)], axis=1)
    table_x = jnp.concatenate(
        [jnp.transpose(x1t), jnp.zeros((n1, 13), f32)], axis=1)

    g1p = _sc_gather(p2r, idx1.reshape(-1))
    g1x = _sc_gather(table_x2, idx1.reshape(-1))
    gx2 = _sc_gather(table_x, idx2.reshape(-1))

    wd3t = jnp.transpose(conv0_w[:, 128:131])
    wd16 = jnp.concatenate([wd3t, jnp.zeros((13, 64), f32)], axis=0)
    wp2 = jnp.transpose(conv0_w[:, 64:128])
    wp1 = jnp.transpose(conv0_w[:, :64])
    w1 = jnp.transpose(conv1_w)

    def wn_prep(w0, w1_, w2):
        w0t = jnp.transpose(w0)
        n0g = jnp.concatenate([w0t, jnp.zeros((13, 8), f32)], axis=0)
        return n0g, jnp.transpose(w1_), jnp.transpose(w2)

    n0g, n1w, n2w = wn_prep(wn1_w0, wn1_w1, wn1_w2)
    m0g, m1w, m2w = wn_prep(wn2_w0, wn2_w1, wn2_w2)

    ppc = _stage1(g1p.reshape(n1, K, 64), g1x.reshape(n1, K, 16), table_x, p1r,
                  wp2, wd16, wp1, conv0_b[None], w1, conv1_b[None],
                  n0g, wn1_b0[None], n1w, wn1_b1[None],
                  n2w, wn1_b2[None])

    gc = _sc_gather(ppc, idx2.reshape(-1))

    out = _stage2(gx2.reshape(n1, K, 16), gc.reshape(n1, K, 64), table_x,
                  m0g, wn2_b0[None], m1w, wn2_b1[None],
                  m2w, wn2_b2[None])
    return jnp.transpose(out)[None]


# speedup vs baseline: 10.3854x; 1.0343x over previous
import functools

import jax
import jax.numpy as jnp
from jax import lax
from jax.experimental import pallas as pl
from jax.experimental.pallas import tpu as pltpu
from jax.experimental.pallas import tpu_sc as plsc

K = 16
LEAK = 0.1
TQ_KNN = 256
TQ_MLP = 512
SC_NC = 2
SC_NS = 16
SC_NW = SC_NC * SC_NS
IMAX = 0x7FFFFFFF


def _knn_body(q_ref, x_ref, out_ref):
    tq = out_ref.shape[0]
    q = q_ref[...]
    x = x_ref[...]
    nx = x.shape[1]
    g = nx // 128
    xsq = jnp.sum(x * x, axis=0, keepdims=True)
    qsq = jnp.sum(q * q, axis=0)
    qx = lax.dot_general(q.astype(jnp.bfloat16), x.astype(jnp.bfloat16),
                         (((0,), (0,)), ((), ())),
                         preferred_element_type=jnp.float32)
    d = (-2.0 * qx + qsq[:, None]) + xsq
    k3 = d.reshape(tq, g, 128)
    liota = lax.broadcasted_iota(jnp.int32, (tq, 128), 1)
    prev = jnp.full((tq, 1), -jnp.inf, jnp.float32)
    cols = []
    for _ in range(K):
        masked = jnp.where(k3 > prev[:, :, None], k3, jnp.inf)
        v2 = jnp.min(masked, axis=1)
        g2 = jnp.argmin(masked, axis=1).astype(jnp.int32)
        kmin = jnp.min(v2, axis=1, keepdims=True)
        flat = jnp.where(v2 == kmin, g2 * 128 + liota, nx)
        idx = jnp.min(flat, axis=1, keepdims=True)
        cols.append(idx)
        prev = kmin
    out_ref[...] = jnp.concatenate(cols, axis=1)


def _knn(q8, x8):
    nq = q8.shape[1]
    nx = x8.shape[1]
    return pl.pallas_call(
        _knn_body,
        grid=(nq // TQ_KNN,),
        in_specs=[pl.BlockSpec((8, TQ_KNN), lambda i: (0, i)),
                  pl.BlockSpec((8, nx), lambda i: (0, 0))],
        out_specs=pl.BlockSpec((TQ_KNN, K), lambda i: (i, 0)),
        out_shape=jax.ShapeDtypeStruct((nq, K), jnp.int32),
    )(q8, x8)


def _sc_gather(table, idx_flat):
    b = idx_flat.shape[0]
    d = table.shape[1]
    r = b // 128
    rw = r // SC_NW
    nb = 4
    ng = rw // nb
    idx2d = idx_flat.reshape(r, 128)
    mesh = plsc.VectorSubcoreMesh(core_axis_name="c", subcore_axis_name="s")

    @functools.partial(
        pl.kernel, mesh=mesh,
        out_type=jax.ShapeDtypeStruct((r, 128, d), jnp.float32),
        compiler_params=pltpu.CompilerParams(use_tc_tiling_on_sc=False),
        scratch_types=[pltpu.VMEM((rw, 128), jnp.int32),
                       pltpu.VMEM((nb, 128, d), jnp.float32),
                       pltpu.SemaphoreType.DMA],
    )
    def gk(table_hbm, idx_hbm, out_hbm, idx_v, rows_v, sem):
        wid = lax.axis_index("s") * SC_NC + lax.axis_index("c")
        r0 = wid * rw
        pltpu.sync_copy(idx_hbm.at[pl.ds(r0, rw)], idx_v)

        def body(gi, carry):
            base = gi * nb
            cps = [pltpu.async_copy(table_hbm.at[idx_v.at[base + j]],
                                    rows_v.at[j], sem)
                   for j in range(nb)]
            for cp in cps:
                cp.wait()
            pltpu.sync_copy(rows_v, out_hbm.at[pl.ds(r0 + base, nb)])
            return carry

        lax.fori_loop(0, ng, body, 0)

    return gk(table, idx2d).reshape(b, d)


def _leaky(v):
    return jnp.where(v >= 0, v, LEAK * v)


def _bdot(a, b):
    return jnp.dot(a.astype(jnp.bfloat16), b.astype(jnp.bfloat16),
                   preferred_element_type=jnp.float32)


def _stage1_body(g_ref, gx_ref, qrow_ref, p1_ref, wp2_ref, wd16_ref, wp1_ref,
                 b0_ref, w1_ref, b1_ref, n0g_ref, nb0_ref,
                 n1_ref, nb1_ref, n2_ref, nb2_ref, out_ref):
    tq = out_ref.shape[0]
    g = g_ref[...].reshape(tq * K, 64)
    gx = gx_ref[...]
    qrow = qrow_ref[...]
    dirp = (gx - qrow[:, None, :]).reshape(tq * K, 16)
    t0 = _bdot(p1_ref[...], wp1_ref[...]) + b0_ref[...]
    z0 = ((_bdot(g, wp2_ref[...]) + _bdot(dirp, wd16_ref[...]))
          .reshape(tq, K, 64) + t0[:, None, :])
    a0 = _leaky(z0).reshape(tq * K, 64)
    z1 = _bdot(a0, w1_ref[...]) + b1_ref[...]
    a1 = _leaky(z1).reshape(tq, K, 64)
    u0 = jnp.maximum(_bdot(dirp, n0g_ref[...]) + nb0_ref[...], 0.0)
    u1 = jnp.maximum(_bdot(u0, n1_ref[...]) + nb1_ref[...], 0.0)
    w = jnp.maximum(_bdot(u1, n2_ref[...]) + nb2_ref[...], 0.0).reshape(tq, K, 64)
    out_ref[...] = jnp.sum(w * a1, axis=1)


def _full(shape):
    return pl.BlockSpec(shape, lambda i: tuple(0 for _ in shape))


def _stage1(g13, gx13, qrow, p1r, wp2, wd16, wp1, b0r, w1, b1r,
            n0g, nb0, n1, nb1, n2, nb2):
    n1q = p1r.shape[0]
    return pl.pallas_call(
        _stage1_body,
        grid=(n1q // TQ_MLP,),
        in_specs=[pl.BlockSpec((TQ_MLP, K, 64), lambda i: (i, 0, 0)),
                  pl.BlockSpec((TQ_MLP, K, 16), lambda i: (i, 0, 0)),
                  pl.BlockSpec((TQ_MLP, 16), lambda i: (i, 0)),
                  pl.BlockSpec((TQ_MLP, 64), lambda i: (i, 0)),
                  _full((64, 64)), _full((16, 64)), _full((64, 64)),
                  _full((1, 64)), _full((64, 64)),
                  _full((1, 64)), _full((16, 8)), _full((1, 8)),
                  _full((8, 8)), _full((1, 8)), _full((8, 64)), _full((1, 64))],
        out_specs=pl.BlockSpec((TQ_MLP, 64), lambda i: (i, 0)),
        out_shape=jax.ShapeDtypeStruct((n1q, 64), jnp.float32),
    )(g13, gx13, qrow, p1r, wp2, wd16, wp1, b0r, w1, b1r,
      n0g, nb0, n1, nb1, n2, nb2)


def _stage2_body(gx_ref, gc_ref, qrow_ref, m0g_ref, mb0_ref,
                 m1_ref, mb1_ref, m2_ref, mb2_ref, out_ref):
    tq = out_ref.shape[0]
    gx = gx_ref[...]
    qrow = qrow_ref[...]
    dirp = (gx - qrow[:, None, :]).reshape(tq * K, 16)
    u0 = jnp.maximum(_bdot(dirp, m0g_ref[...]) + mb0_ref[...], 0.0)
    u1 = jnp.maximum(_bdot(u0, m1_ref[...]) + mb1_ref[...], 0.0)
    w = jnp.maximum(_bdot(u1, m2_ref[...]) + mb2_ref[...], 0.0).reshape(tq, K, 64)
    out_ref[...] = jnp.sum(w * gc_ref[...], axis=1)


def _stage2(gx3, gc3, qrow, m0g, mb0, m1, mb1, m2, mb2):
    n1q = qrow.shape[0]
    return pl.pallas_call(
        _stage2_body,
        grid=(n1q // TQ_MLP,),
        in_specs=[pl.BlockSpec((TQ_MLP, K, 16), lambda i: (i, 0, 0)),
                  pl.BlockSpec((TQ_MLP, K, 64), lambda i: (i, 0, 0)),
                  pl.BlockSpec((TQ_MLP, 16), lambda i: (i, 0)),
                  _full((16, 8)), _full((1, 8)),
                  _full((8, 8)), _full((1, 8)), _full((8, 64)), _full((1, 64))],
        out_specs=pl.BlockSpec((TQ_MLP, 64), lambda i: (i, 0)),
        out_shape=jax.ShapeDtypeStruct((n1q, 64), jnp.float32),
    )(gx3, gc3, qrow, m0g, mb0, m1, mb1, m2, mb2)


def kernel(xyz1, xyz2, points1, points2, conv0_w, conv0_b, conv1_w, conv1_b,
           wn1_w0, wn1_b0, wn1_w1, wn1_b1, wn1_w2, wn1_b2,
           wn2_w0, wn2_b0, wn2_w1, wn2_b1, wn2_w2, wn2_b2):
    n1 = xyz1.shape[2]
    n2 = xyz2.shape[2]
    f32 = jnp.float32
    x1t = xyz1[0]
    x2t = xyz2[0]
    p1r = jnp.transpose(points1[0])
    p2r = jnp.transpose(points2[0])
    x1p8 = jnp.concatenate([x1t, jnp.zeros((5, n1), f32)], axis=0)
    x2p8 = jnp.concatenate([x2t, jnp.zeros((5, n2), f32)], axis=0)

    idx1 = _knn(x1p8, x2p8)
    idx2 = _knn(x1p8, x1p8)

    table_x2 = jnp.concatenate(
        [jnp.transpose(x2t), jnp.zeros((n2, 13), f32)], axis=1)
    table_x = jnp.concatenate(
        [jnp.transpose(x1t), jnp.zeros((n1, 13), f32)], axis=1)

    g1p = _sc_gather(p2r, idx1.reshape(-1))
    g1x = _sc_gather(table_x2, idx1.reshape(-1))
    gx2 = _sc_gather(table_x, idx2.reshape(-1))

    wd3t = jnp.transpose(conv0_w[:, 128:131])
    wd16 = jnp.concatenate([wd3t, jnp.zeros((13, 64), f32)], axis=0)
    wp2 = jnp.transpose(conv0_w[:, 64:128])
    wp1 = jnp.transpose(conv0_w[:, :64])
    w1 = jnp.transpose(conv1_w)

    def wn_prep(w0, w1_, w2):
        w0t = jnp.transpose(w0)
        n0g = jnp.concatenate([w0t, jnp.zeros((13, 8), f32)], axis=0)
        return n0g, jnp.transpose(w1_), jnp.transpose(w2)

    n0g, n1w, n2w = wn_prep(wn1_w0, wn1_w1, wn1_w2)
    m0g, m1w, m2w = wn_prep(wn2_w0, wn2_w1, wn2_w2)

    ppc = _stage1(g1p.reshape(n1, K, 64), g1x.reshape(n1, K, 16), table_x, p1r,
                  wp2, wd16, wp1, conv0_b[None], w1, conv1_b[None],
                  n0g, wn1_b0[None], n1w, wn1_b1[None],
                  n2w, wn1_b2[None])

    gc = _sc_gather(ppc, idx2.reshape(-1))

    out = _stage2(gx2.reshape(n1, K, 16), gc.reshape(n1, K, 64), table_x,
                  m0g, wn2_b0[None], m1w, wn2_b1[None],
                  m2w, wn2_b2[None])
    return jnp.transpose(out)[None]


# speedup vs baseline: 11.2449x; 1.0828x over previous
import functools

import jax
import jax.numpy as jnp
from jax import lax
from jax.experimental import pallas as pl
from jax.experimental.pallas import tpu as pltpu
from jax.experimental.pallas import tpu_sc as plsc

K = 16
LEAK = 0.1
TQ_KNN = 512
TQ_MLP = 512
SC_NC = 2
SC_NS = 16
SC_NW = SC_NC * SC_NS
IMAX = 0x7FFFFFFF


def _knn_body(q_ref, x_ref, out_ref):
    tq = out_ref.shape[0]
    q = q_ref[...]
    x = x_ref[...]
    nx = x.shape[1]
    g = nx // 128
    xsq = jnp.sum(x * x, axis=0, keepdims=True)
    qsq = jnp.sum(q * q, axis=0)
    qx = lax.dot_general(q.astype(jnp.bfloat16), x.astype(jnp.bfloat16),
                         (((0,), (0,)), ((), ())),
                         preferred_element_type=jnp.float32)
    d = (-2.0 * qx + qsq[:, None]) + xsq
    k3 = d.reshape(tq, g, 128)
    liota = lax.broadcasted_iota(jnp.int32, (tq, 128), 1)
    prev = jnp.full((tq, 1), -jnp.inf, jnp.float32)
    cols = []
    for _ in range(K):
        masked = jnp.where(k3 > prev[:, :, None], k3, jnp.inf)
        v2 = jnp.min(masked, axis=1)
        g2 = jnp.argmin(masked, axis=1).astype(jnp.int32)
        kmin = jnp.min(v2, axis=1, keepdims=True)
        flat = jnp.where(v2 == kmin, g2 * 128 + liota, nx)
        idx = jnp.min(flat, axis=1, keepdims=True)
        cols.append(idx)
        prev = kmin
    out_ref[...] = jnp.concatenate(cols, axis=1)


def _knn(q8, x8):
    nq = q8.shape[1]
    nx = x8.shape[1]
    return pl.pallas_call(
        _knn_body,
        grid=(nq // TQ_KNN,),
        in_specs=[pl.BlockSpec((8, TQ_KNN), lambda i: (0, i)),
                  pl.BlockSpec((8, nx), lambda i: (0, 0))],
        out_specs=pl.BlockSpec((TQ_KNN, K), lambda i: (i, 0)),
        out_shape=jax.ShapeDtypeStruct((nq, K), jnp.int32),
    )(q8, x8)


def _sc_gather(table, idx_flat):
    b = idx_flat.shape[0]
    d = table.shape[1]
    r = b // 128
    rw = r // SC_NW
    nb = 4
    ng = rw // nb
    idx2d = idx_flat.reshape(r, 128)
    mesh = plsc.VectorSubcoreMesh(core_axis_name="c", subcore_axis_name="s")

    @functools.partial(
        pl.kernel, mesh=mesh,
        out_type=jax.ShapeDtypeStruct((r, 128, d), jnp.float32),
        compiler_params=pltpu.CompilerParams(use_tc_tiling_on_sc=False),
        scratch_types=[pltpu.VMEM((rw, 128), jnp.int32),
                       pltpu.VMEM((nb, 128, d), jnp.float32),
                       pltpu.SemaphoreType.DMA],
    )
    def gk(table_hbm, idx_hbm, out_hbm, idx_v, rows_v, sem):
        wid = lax.axis_index("s") * SC_NC + lax.axis_index("c")
        r0 = wid * rw
        pltpu.sync_copy(idx_hbm.at[pl.ds(r0, rw)], idx_v)

        def body(gi, carry):
            base = gi * nb
            cps = [pltpu.async_copy(table_hbm.at[idx_v.at[base + j]],
                                    rows_v.at[j], sem)
                   for j in range(nb)]
            for cp in cps:
                cp.wait()
            pltpu.sync_copy(rows_v, out_hbm.at[pl.ds(r0 + base, nb)])
            return carry

        lax.fori_loop(0, ng, body, 0)

    return gk(table, idx2d).reshape(b, d)


def _leaky(v):
    return jnp.where(v >= 0, v, LEAK * v)


def _bdot(a, b):
    return jnp.dot(a.astype(jnp.bfloat16), b.astype(jnp.bfloat16),
                   preferred_element_type=jnp.float32)


def _stage1_body(g_ref, gx_ref, qrow_ref, p1_ref, wp2_ref, wd16_ref, wp1_ref,
                 b0_ref, w1_ref, b1_ref, n0g_ref, nb0_ref,
                 n1_ref, nb1_ref, n2_ref, nb2_ref, out_ref):
    tq = out_ref.shape[0]
    g = g_ref[...].reshape(tq * K, 64)
    gx = gx_ref[...]
    qrow = qrow_ref[...]
    dirp = (gx - qrow[:, None, :]).reshape(tq * K, 16)
    t0 = _bdot(p1_ref[...], wp1_ref[...]) + b0_ref[...]
    z0 = ((_bdot(g, wp2_ref[...]) + _bdot(dirp, wd16_ref[...]))
          .reshape(tq, K, 64) + t0[:, None, :])
    a0 = _leaky(z0).reshape(tq * K, 64)
    z1 = _bdot(a0, w1_ref[...]) + b1_ref[...]
    a1 = _leaky(z1).reshape(tq, K, 64)
    u0 = jnp.maximum(_bdot(dirp, n0g_ref[...]) + nb0_ref[...], 0.0)
    u1 = jnp.maximum(_bdot(u0, n1_ref[...]) + nb1_ref[...], 0.0)
    w = jnp.maximum(_bdot(u1, n2_ref[...]) + nb2_ref[...], 0.0).reshape(tq, K, 64)
    out_ref[...] = jnp.sum(w * a1, axis=1)


def _full(shape):
    return pl.BlockSpec(shape, lambda i: tuple(0 for _ in shape))


def _stage1(g13, gx13, qrow, p1r, wp2, wd16, wp1, b0r, w1, b1r,
            n0g, nb0, n1, nb1, n2, nb2):
    n1q = p1r.shape[0]
    return pl.pallas_call(
        _stage1_body,
        grid=(n1q // TQ_MLP,),
        in_specs=[pl.BlockSpec((TQ_MLP, K, 64), lambda i: (i, 0, 0)),
                  pl.BlockSpec((TQ_MLP, K, 16), lambda i: (i, 0, 0)),
                  pl.BlockSpec((TQ_MLP, 16), lambda i: (i, 0)),
                  pl.BlockSpec((TQ_MLP, 64), lambda i: (i, 0)),
                  _full((64, 64)), _full((16, 64)), _full((64, 64)),
                  _full((1, 64)), _full((64, 64)),
                  _full((1, 64)), _full((16, 8)), _full((1, 8)),
                  _full((8, 8)), _full((1, 8)), _full((8, 64)), _full((1, 64))],
        out_specs=pl.BlockSpec((TQ_MLP, 64), lambda i: (i, 0)),
        out_shape=jax.ShapeDtypeStruct((n1q, 64), jnp.float32),
    )(g13, gx13, qrow, p1r, wp2, wd16, wp1, b0r, w1, b1r,
      n0g, nb0, n1, nb1, n2, nb2)


def _stage2_body(gx_ref, gc_ref, qrow_ref, m0g_ref, mb0_ref,
                 m1_ref, mb1_ref, m2_ref, mb2_ref, out_ref):
    tq = out_ref.shape[0]
    gx = gx_ref[...]
    qrow = qrow_ref[...]
    dirp = (gx - qrow[:, None, :]).reshape(tq * K, 16)
    u0 = jnp.maximum(_bdot(dirp, m0g_ref[...]) + mb0_ref[...], 0.0)
    u1 = jnp.maximum(_bdot(u0, m1_ref[...]) + mb1_ref[...], 0.0)
    w = jnp.maximum(_bdot(u1, m2_ref[...]) + mb2_ref[...], 0.0).reshape(tq, K, 64)
    out_ref[...] = jnp.sum(w * gc_ref[...], axis=1)


def _stage2(gx3, gc3, qrow, m0g, mb0, m1, mb1, m2, mb2):
    n1q = qrow.shape[0]
    return pl.pallas_call(
        _stage2_body,
        grid=(n1q // TQ_MLP,),
        in_specs=[pl.BlockSpec((TQ_MLP, K, 16), lambda i: (i, 0, 0)),
                  pl.BlockSpec((TQ_MLP, K, 64), lambda i: (i, 0, 0)),
                  pl.BlockSpec((TQ_MLP, 16), lambda i: (i, 0)),
                  _full((16, 8)), _full((1, 8)),
                  _full((8, 8)), _full((1, 8)), _full((8, 64)), _full((1, 64))],
        out_specs=pl.BlockSpec((TQ_MLP, 64), lambda i: (i, 0)),
        out_shape=jax.ShapeDtypeStruct((n1q, 64), jnp.float32),
    )(gx3, gc3, qrow, m0g, mb0, m1, mb1, m2, mb2)


def kernel(xyz1, xyz2, points1, points2, conv0_w, conv0_b, conv1_w, conv1_b,
           wn1_w0, wn1_b0, wn1_w1, wn1_b1, wn1_w2, wn1_b2,
           wn2_w0, wn2_b0, wn2_w1, wn2_b1, wn2_w2, wn2_b2):
    n1 = xyz1.shape[2]
    n2 = xyz2.shape[2]
    f32 = jnp.float32
    x1t = xyz1[0]
    x2t = xyz2[0]
    p1r = jnp.transpose(points1[0])
    p2r = jnp.transpose(points2[0])
    x1p8 = jnp.concatenate([x1t, jnp.zeros((5, n1), f32)], axis=0)
    x2p8 = jnp.concatenate([x2t, jnp.zeros((5, n2), f32)], axis=0)

    idx1 = _knn(x1p8, x2p8)
    idx2 = _knn(x1p8, x1p8)

    table_x2 = jnp.concatenate(
        [jnp.transpose(x2t), jnp.zeros((n2, 13), f32)], axis=1)
    table_x = jnp.concatenate(
        [jnp.transpose(x1t), jnp.zeros((n1, 13), f32)], axis=1)

    g1p = _sc_gather(p2r, idx1.reshape(-1))
    g1x = _sc_gather(table_x2, idx1.reshape(-1))
    gx2 = _sc_gather(table_x, idx2.reshape(-1))

    wd3t = jnp.transpose(conv0_w[:, 128:131])
    wd16 = jnp.concatenate([wd3t, jnp.zeros((13, 64), f32)], axis=0)
    wp2 = jnp.transpose(conv0_w[:, 64:128])
    wp1 = jnp.transpose(conv0_w[:, :64])
    w1 = jnp.transpose(conv1_w)

    def wn_prep(w0, w1_, w2):
        w0t = jnp.transpose(w0)
        n0g = jnp.concatenate([w0t, jnp.zeros((13, 8), f32)], axis=0)
        return n0g, jnp.transpose(w1_), jnp.transpose(w2)

    n0g, n1w, n2w = wn_prep(wn1_w0, wn1_w1, wn1_w2)
    m0g, m1w, m2w = wn_prep(wn2_w0, wn2_w1, wn2_w2)

    ppc = _stage1(g1p.reshape(n1, K, 64), g1x.reshape(n1, K, 16), table_x, p1r,
                  wp2, wd16, wp1, conv0_b[None], w1, conv1_b[None],
                  n0g, wn1_b0[None], n1w, wn1_b1[None],
                  n2w, wn1_b2[None])

    gc = _sc_gather(ppc, idx2.reshape(-1))

    out = _stage2(gx2.reshape(n1, K, 16), gc.reshape(n1, K, 64), table_x,
                  m0g, wn2_b0[None], m1w, wn2_b1[None],
                  m2w, wn2_b2[None])
    return jnp.transpose(out)[None]


# speedup vs baseline: 11.2462x; 1.0001x over previous
import functools

import jax
import jax.numpy as jnp
from jax import lax
from jax.experimental import pallas as pl
from jax.experimental.pallas import tpu as pltpu
from jax.experimental.pallas import tpu_sc as plsc

K = 16
LEAK = 0.1
TQ_KNN = 512
TQ_MLP = 512
SC_NC = 2
SC_NS = 16
SC_NW = SC_NC * SC_NS


def _knn_body(q_ref, x_ref, out_ref):
    tq = out_ref.shape[0]
    q = q_ref[...]
    x = x_ref[...]
    nx = x.shape[1]
    g = nx // 128
    xsq = jnp.sum(x * x, axis=0, keepdims=True)
    qsq = jnp.sum(q * q, axis=0)
    qx = lax.dot_general(q.astype(jnp.bfloat16), x.astype(jnp.bfloat16),
                         (((0,), (0,)), ((), ())),
                         preferred_element_type=jnp.float32)
    d = (-2.0 * qx + qsq[:, None]) + xsq
    k3 = d.reshape(tq, g, 128)
    liota = lax.broadcasted_iota(jnp.int32, (tq, 128), 1)
    prev = jnp.full((tq, 1), -jnp.inf, jnp.float32)
    cols = []
    for _ in range(K):
        masked = jnp.where(k3 > prev[:, :, None], k3, jnp.inf)
        v2 = jnp.min(masked, axis=1)
        g2 = jnp.argmin(masked, axis=1).astype(jnp.int32)
        kmin = jnp.min(v2, axis=1, keepdims=True)
        flat = jnp.where(v2 == kmin, g2 * 128 + liota, nx)
        idx = jnp.min(flat, axis=1, keepdims=True)
        cols.append(idx)
        prev = kmin
    out_ref[...] = jnp.concatenate(cols, axis=1)


def _knn(q8, x8):
    nq = q8.shape[1]
    nx = x8.shape[1]
    return pl.pallas_call(
        _knn_body,
        grid=(nq // TQ_KNN,),
        in_specs=[pl.BlockSpec((8, TQ_KNN), lambda i: (0, i)),
                  pl.BlockSpec((8, nx), lambda i: (0, 0))],
        out_specs=pl.BlockSpec((TQ_KNN, K), lambda i: (i, 0)),
        out_shape=jax.ShapeDtypeStruct((nq, K), jnp.int32),
    )(q8, x8)


def _sc_gather(table, idx_flat):
    b = idx_flat.shape[0]
    d = table.shape[1]
    r = b // 128
    rw = r // SC_NW
    nb = 4
    ng = rw // nb
    idx2d = idx_flat.reshape(r, 128)
    mesh = plsc.VectorSubcoreMesh(core_axis_name="c", subcore_axis_name="s")

    @functools.partial(
        pl.kernel, mesh=mesh,
        out_type=jax.ShapeDtypeStruct((r, 128, d), jnp.float32),
        compiler_params=pltpu.CompilerParams(use_tc_tiling_on_sc=False),
        scratch_types=[pltpu.VMEM((rw, 128), jnp.int32),
                       pltpu.VMEM((nb, 128, d), jnp.float32),
                       pltpu.SemaphoreType.DMA],
    )
    def gk(table_hbm, idx_hbm, out_hbm, idx_v, rows_v, sem):
        wid = lax.axis_index("s") * SC_NC + lax.axis_index("c")
        r0 = wid * rw
        pltpu.sync_copy(idx_hbm.at[pl.ds(r0, rw)], idx_v)

        def body(gi, carry):
            base = gi * nb
            cps = [pltpu.async_copy(table_hbm.at[idx_v.at[base + j]],
                                    rows_v.at[j], sem)
                   for j in range(nb)]
            for cp in cps:
                cp.wait()
            pltpu.sync_copy(rows_v, out_hbm.at[pl.ds(r0 + base, nb)])
            return carry

        lax.fori_loop(0, ng, body, 0)

    return gk(table, idx2d).reshape(b, d)


def _leaky(v):
    return jnp.where(v >= 0, v, LEAK * v)


def _bdot(a, b):
    return jnp.dot(a.astype(jnp.bfloat16), b.astype(jnp.bfloat16),
                   preferred_element_type=jnp.float32)


def _stage1_body(g_ref, gx_ref, qrow_ref, p1_ref, wp2_ref, wd16_ref, wp1_ref,
                 b0_ref, w1_ref, b1_ref, n0g_ref, nb0_ref,
                 n1_ref, nb1_ref, n2_ref, nb2_ref, out_ref):
    tq = out_ref.shape[0]
    g = g_ref[...].reshape(tq * K, 64)
    gx = gx_ref[...]
    qrow = qrow_ref[...]
    dirp = (gx - qrow[:, None, :]).reshape(tq * K, 16)
    t0 = _bdot(p1_ref[...], wp1_ref[...]) + b0_ref[...]
    z0 = ((_bdot(g, wp2_ref[...]) + _bdot(dirp, wd16_ref[...]))
          .reshape(tq, K, 64) + t0[:, None, :])
    a0 = _leaky(z0).reshape(tq * K, 64)
    z1 = _bdot(a0, w1_ref[...]) + b1_ref[...]
    a1 = _leaky(z1).reshape(tq, K, 64)
    u0 = jnp.maximum(_bdot(dirp, n0g_ref[...]) + nb0_ref[...], 0.0)
    u1 = jnp.maximum(_bdot(u0, n1_ref[...]) + nb1_ref[...], 0.0)
    w = jnp.maximum(_bdot(u1, n2_ref[...]) + nb2_ref[...], 0.0).reshape(tq, K, 64)
    out_ref[...] = jnp.sum(w * a1, axis=1)


def _full(shape):
    return pl.BlockSpec(shape, lambda i: tuple(0 for _ in shape))


def _stage1(g13, gx13, qrow, p1r, wp2, wd16, wp1, b0r, w1, b1r,
            n0g, nb0, n1, nb1, n2, nb2):
    n1q = p1r.shape[0]
    return pl.pallas_call(
        _stage1_body,
        grid=(n1q // TQ_MLP,),
        in_specs=[pl.BlockSpec((TQ_MLP, K, 64), lambda i: (i, 0, 0)),
                  pl.BlockSpec((TQ_MLP, K, 16), lambda i: (i, 0, 0)),
                  pl.BlockSpec((TQ_MLP, 16), lambda i: (i, 0)),
                  pl.BlockSpec((TQ_MLP, 64), lambda i: (i, 0)),
                  _full((64, 64)), _full((16, 64)), _full((64, 64)),
                  _full((1, 64)), _full((64, 64)),
                  _full((1, 64)), _full((16, 8)), _full((1, 8)),
                  _full((8, 8)), _full((1, 8)), _full((8, 64)), _full((1, 64))],
        out_specs=pl.BlockSpec((TQ_MLP, 64), lambda i: (i, 0)),
        out_shape=jax.ShapeDtypeStruct((n1q, 64), jnp.float32),
    )(g13, gx13, qrow, p1r, wp2, wd16, wp1, b0r, w1, b1r,
      n0g, nb0, n1, nb1, n2, nb2)


def _stage2_body(gx_ref, gc_ref, qrow_ref, m0g_ref, mb0_ref,
                 m1_ref, mb1_ref, m2_ref, mb2_ref, out_ref):
    tq = out_ref.shape[0]
    gx = gx_ref[...]
    qrow = qrow_ref[...]
    dirp = (gx - qrow[:, None, :]).reshape(tq * K, 16)
    u0 = jnp.maximum(_bdot(dirp, m0g_ref[...]) + mb0_ref[...], 0.0)
    u1 = jnp.maximum(_bdot(u0, m1_ref[...]) + mb1_ref[...], 0.0)
    w = jnp.maximum(_bdot(u1, m2_ref[...]) + mb2_ref[...], 0.0).reshape(tq, K, 64)
    out_ref[...] = jnp.sum(w * gc_ref[...], axis=1)


def _stage2(gx3, gc3, qrow, m0g, mb0, m1, mb1, m2, mb2):
    n1q = qrow.shape[0]
    return pl.pallas_call(
        _stage2_body,
        grid=(n1q // TQ_MLP,),
        in_specs=[pl.BlockSpec((TQ_MLP, K, 16), lambda i: (i, 0, 0)),
                  pl.BlockSpec((TQ_MLP, K, 64), lambda i: (i, 0, 0)),
                  pl.BlockSpec((TQ_MLP, 16), lambda i: (i, 0)),
                  _full((16, 8)), _full((1, 8)),
                  _full((8, 8)), _full((1, 8)), _full((8, 64)), _full((1, 64))],
        out_specs=pl.BlockSpec((TQ_MLP, 64), lambda i: (i, 0)),
        out_shape=jax.ShapeDtypeStruct((n1q, 64), jnp.float32),
    )(gx3, gc3, qrow, m0g, mb0, m1, mb1, m2, mb2)


def kernel(xyz1, xyz2, points1, points2, conv0_w, conv0_b, conv1_w, conv1_b,
           wn1_w0, wn1_b0, wn1_w1, wn1_b1, wn1_w2, wn1_b2,
           wn2_w0, wn2_b0, wn2_w1, wn2_b1, wn2_w2, wn2_b2):
    n1 = xyz1.shape[2]
    n2 = xyz2.shape[2]
    f32 = jnp.float32
    x1t = xyz1[0]
    x2t = xyz2[0]
    p1r = jnp.transpose(points1[0])
    p2r = jnp.transpose(points2[0])
    x1p8 = jnp.concatenate([x1t, jnp.zeros((5, n1), f32)], axis=0)
    x2p8 = jnp.concatenate([x2t, jnp.zeros((5, n2), f32)], axis=0)

    idx1 = _knn(x1p8, x2p8)
    idx2 = _knn(x1p8, x1p8)

    table_x2 = jnp.concatenate(
        [jnp.transpose(x2t), jnp.zeros((n2, 13), f32)], axis=1)
    table_x = jnp.concatenate(
        [jnp.transpose(x1t), jnp.zeros((n1, 13), f32)], axis=1)

    g1p = _sc_gather(p2r, idx1.reshape(-1))
    g1x = _sc_gather(table_x2, idx1.reshape(-1))
    gx2 = _sc_gather(table_x, idx2.reshape(-1))

    wd3t = jnp.transpose(conv0_w[:, 128:131])
    wd16 = jnp.concatenate([wd3t, jnp.zeros((13, 64), f32)], axis=0)
    wp2 = jnp.transpose(conv0_w[:, 64:128])
    wp1 = jnp.transpose(conv0_w[:, :64])
    w1 = jnp.transpose(conv1_w)

    def wn_prep(w0, w1_, w2):
        w0t = jnp.transpose(w0)
        n0g = jnp.concatenate([w0t, jnp.zeros((13, 8), f32)], axis=0)
        return n0g, jnp.transpose(w1_), jnp.transpose(w2)

    n0g, n1w, n2w = wn_prep(wn1_w0, wn1_w1, wn1_w2)
    m0g, m1w, m2w = wn_prep(wn2_w0, wn2_w1, wn2_w2)

    ppc = _stage1(g1p.reshape(n1, K, 64), g1x.reshape(n1, K, 16), table_x, p1r,
                  wp2, wd16, wp1, conv0_b[None], w1, conv1_b[None],
                  n0g, wn1_b0[None], n1w, wn1_b1[None],
                  n2w, wn1_b2[None])

    gc = _sc_gather(ppc, idx2.reshape(-1))

    out = _stage2(gx2.reshape(n1, K, 16), gc.reshape(n1, K, 64), table_x,
                  m0g, wn2_b0[None], m1w, wn2_b1[None],
                  m2w, wn2_b2[None])
    return jnp.transpose(out)[None]
